# baseline (device time: 16778 ns/iter reference)
import jax
import jax.numpy as jnp
from jax import lax
from jax.experimental import pallas as pl
from jax.experimental.pallas import tpu as pltpu

K_SLOTS = 320
N_CHUNKS = 4


def kernel(ids, E):
    T = ids.shape[0]
    V_local, D = E.shape
    KC = K_SLOTS // N_CHUNKS

    ids2 = ids.reshape(T, 1)
    ids_row = ids.reshape(1, T)

    def body(ids_ref, idsr_ref, e_hbm, out_ref, evmem, cbuf, rbuf,
             copy_sem, send_sems, recv_sems):
        my_x = lax.axis_index("x")
        my_y = lax.axis_index("y")
        my_z = lax.axis_index("z")
        y_partner = (my_x, 1 - my_y, my_z)

        ecopy = pltpu.make_async_copy(e_hbm, evmem, copy_sem)
        ecopy.start()

        barrier = pltpu.get_barrier_semaphore()
        pl.semaphore_signal(
            barrier, inc=1, device_id=y_partner,
            device_id_type=pl.DeviceIdType.MESH,
        )
        pl.semaphore_wait(barrier, 1)

        idsv = ids_ref[:, :]
        local_m = idsv - my_y * V_local
        in_m = jnp.logical_and(local_m >= 0, local_m < V_local)
        in_o = jnp.logical_not(in_m)
        m2 = jnp.concatenate(
            [in_m.astype(jnp.bfloat16), in_o.astype(jnp.bfloat16)], axis=1
        )

        tri = (
            lax.broadcasted_iota(jnp.int32, (T, T), 0)
            > lax.broadcasted_iota(jnp.int32, (T, T), 1)
        ).astype(jnp.bfloat16)
        ranks = jnp.dot(
            tri, m2, preferred_element_type=jnp.float32
        ).astype(jnp.int32)
        r_m = ranks[:, 0:1]
        r_o = ranks[:, 1:2]

        local_mT = idsr_ref[:, :] - my_y * V_local
        in_mT = jnp.logical_and(local_mT >= 0, local_mT < V_local)
        triu = (
            lax.broadcasted_iota(jnp.int32, (T, T), 0)
            < lax.broadcasted_iota(jnp.int32, (T, T), 1)
        ).astype(jnp.bfloat16)
        r_mT = jnp.dot(
            in_mT.astype(jnp.bfloat16), triu,
            preferred_element_type=jnp.float32,
        ).astype(jnp.int32)

        iota_tk = lax.broadcasted_iota(jnp.int32, (T, K_SLOTS), 1)
        st_m = jnp.logical_and(iota_tk == r_m, in_m)
        st_o = jnp.logical_and(iota_tk == r_o, in_o)

        iota_kt = lax.broadcasted_iota(jnp.int32, (K_SLOTS, T), 0)
        st_mT = jnp.logical_and(iota_kt == r_mT, in_mT)
        masked = jnp.where(in_m, local_m, 0)
        digits = jnp.concatenate(
            [(masked // 64).astype(jnp.bfloat16),
             (masked % 64).astype(jnp.bfloat16)], axis=1)
        c2 = jnp.dot(
            st_mT.astype(jnp.bfloat16), digits,
            preferred_element_type=jnp.float32,
        )
        c_local = (c2[:, 0:1] * 64.0 + c2[:, 1:2]).astype(jnp.int32)

        ecopy.wait()
        e_bf16 = evmem[:, :].astype(jnp.bfloat16)
        rdmas = []
        for q in range(N_CHUNKS):
            rows = pl.ds(q * KC, KC)
            iota_kv = lax.broadcasted_iota(jnp.int32, (KC, V_local), 1)
            onehot = (iota_kv == c_local[q * KC:(q + 1) * KC, :]).astype(
                jnp.bfloat16
            )
            crows = jnp.dot(
                onehot, e_bf16, preferred_element_type=jnp.float32
            )
            cbuf[rows, :] = crows.astype(jnp.bfloat16)
            rdma = pltpu.make_async_remote_copy(
                src_ref=cbuf.at[rows],
                dst_ref=rbuf.at[rows],
                send_sem=send_sems.at[q],
                recv_sem=recv_sems.at[q],
                device_id=y_partner,
                device_id_type=pl.DeviceIdType.MESH,
            )
            rdma.start()
            rdmas.append(rdma)

        out_ref[:, :] = jnp.dot(
            st_m.astype(jnp.bfloat16), cbuf[:, :],
            preferred_element_type=jnp.float32,
        ).astype(jnp.bfloat16)

        st_o16 = st_o.astype(jnp.bfloat16)
        for q, rdma in enumerate(rdmas):
            rows = pl.ds(q * KC, KC)
            rdma.wait_recv()
            out_ref[:, :] += jnp.dot(
                st_o16[:, q * KC:(q + 1) * KC], rbuf[rows, :],
                preferred_element_type=jnp.float32,
            ).astype(jnp.bfloat16)
        for rdma in rdmas:
            rdma.wait_send()

    return pl.pallas_call(
        body,
        out_shape=jax.ShapeDtypeStruct((T, D), jnp.bfloat16),
        in_specs=[
            pl.BlockSpec(memory_space=pltpu.VMEM),
            pl.BlockSpec(memory_space=pltpu.VMEM),
            pl.BlockSpec(memory_space=pl.ANY),
        ],
        out_specs=pl.BlockSpec(memory_space=pltpu.VMEM),
        scratch_shapes=[
            pltpu.VMEM((V_local, D), jnp.float32),
            pltpu.VMEM((K_SLOTS, D), jnp.bfloat16),
            pltpu.VMEM((K_SLOTS, D), jnp.bfloat16),
            pltpu.SemaphoreType.DMA,
            pltpu.SemaphoreType.DMA((N_CHUNKS,)),
            pltpu.SemaphoreType.DMA((N_CHUNKS,)),
        ],
        compiler_params=pltpu.CompilerParams(collective_id=0),
    )(ids2, ids_row, E)


# device time: 15818 ns/iter; 1.0607x vs baseline; 1.0607x over previous
import jax
import jax.numpy as jnp
from jax import lax
from jax.experimental import pallas as pl
from jax.experimental.pallas import tpu as pltpu

K_SLOTS = 320
N_CHUNKS = 2


def kernel(ids, E):
    T = ids.shape[0]
    V_local, D = E.shape
    KC = K_SLOTS // N_CHUNKS

    ids2 = ids.reshape(T, 1)
    ids_row = ids.reshape(1, T)

    def body(ids_ref, idsr_ref, e_ref, out_ref, cbuf, rbuf, send_sems,
             recv_sems):
        my_x = lax.axis_index("x")
        my_y = lax.axis_index("y")
        my_z = lax.axis_index("z")
        y_partner = (my_x, 1 - my_y, my_z)

        barrier = pltpu.get_barrier_semaphore()
        pl.semaphore_signal(
            barrier, inc=1, device_id=y_partner,
            device_id_type=pl.DeviceIdType.MESH,
        )
        pl.semaphore_wait(barrier, 1)

        idsv = ids_ref[:, :]
        local_m = idsv - my_y * V_local
        in_m = jnp.logical_and(local_m >= 0, local_m < V_local)
        in_o = jnp.logical_not(in_m)
        m2 = jnp.concatenate(
            [in_m.astype(jnp.bfloat16), in_o.astype(jnp.bfloat16)], axis=1
        )

        tri = (
            lax.broadcasted_iota(jnp.int32, (T, T), 0)
            > lax.broadcasted_iota(jnp.int32, (T, T), 1)
        ).astype(jnp.bfloat16)
        ranks = jnp.dot(
            tri, m2, preferred_element_type=jnp.float32
        ).astype(jnp.int32)
        r_m = ranks[:, 0:1]
        r_o = ranks[:, 1:2]

        local_mT = idsr_ref[:, :] - my_y * V_local
        in_mT = jnp.logical_and(local_mT >= 0, local_mT < V_local)
        triu = (
            lax.broadcasted_iota(jnp.int32, (T, T), 0)
            < lax.broadcasted_iota(jnp.int32, (T, T), 1)
        ).astype(jnp.bfloat16)
        r_mT = jnp.dot(
            in_mT.astype(jnp.bfloat16), triu,
            preferred_element_type=jnp.float32,
        ).astype(jnp.int32)

        iota_tk = lax.broadcasted_iota(jnp.int32, (T, K_SLOTS), 1)
        st_m = jnp.logical_and(iota_tk == r_m, in_m)
        st_o = jnp.logical_and(iota_tk == r_o, in_o)

        iota_kt = lax.broadcasted_iota(jnp.int32, (K_SLOTS, T), 0)
        st_mT = jnp.logical_and(iota_kt == r_mT, in_mT)
        masked = jnp.where(in_m, local_m, 0)
        digits = jnp.concatenate(
            [(masked // 64).astype(jnp.bfloat16),
             (masked % 64).astype(jnp.bfloat16)], axis=1)
        c2 = jnp.dot(
            st_mT.astype(jnp.bfloat16), digits,
            preferred_element_type=jnp.float32,
        )
        c_local = (c2[:, 0:1] * 64.0 + c2[:, 1:2]).astype(jnp.int32)

        e_bf16 = e_ref[:, :].astype(jnp.bfloat16)
        rdmas = []
        for q in range(N_CHUNKS):
            rows = pl.ds(q * KC, KC)
            iota_kv = lax.broadcasted_iota(jnp.int32, (KC, V_local), 1)
            onehot = (iota_kv == c_local[q * KC:(q + 1) * KC, :]).astype(
                jnp.bfloat16
            )
            crows = jnp.dot(
                onehot, e_bf16, preferred_element_type=jnp.float32
            )
            cbuf[rows, :] = crows.astype(jnp.bfloat16)
            rdma = pltpu.make_async_remote_copy(
                src_ref=cbuf.at[rows],
                dst_ref=rbuf.at[rows],
                send_sem=send_sems.at[q],
                recv_sem=recv_sems.at[q],
                device_id=y_partner,
                device_id_type=pl.DeviceIdType.MESH,
            )
            rdma.start()
            rdmas.append(rdma)

        out_ref[:, :] = jnp.dot(
            st_m.astype(jnp.bfloat16), cbuf[:, :],
            preferred_element_type=jnp.float32,
        ).astype(jnp.bfloat16)

        st_o16 = st_o.astype(jnp.bfloat16)
        for q, rdma in enumerate(rdmas):
            rows = pl.ds(q * KC, KC)
            rdma.wait_recv()
            out_ref[:, :] += jnp.dot(
                st_o16[:, q * KC:(q + 1) * KC], rbuf[rows, :],
                preferred_element_type=jnp.float32,
            ).astype(jnp.bfloat16)
        for rdma in rdmas:
            rdma.wait_send()

    return pl.pallas_call(
        body,
        out_shape=jax.ShapeDtypeStruct((T, D), jnp.bfloat16),
        in_specs=[
            pl.BlockSpec(memory_space=pltpu.VMEM),
            pl.BlockSpec(memory_space=pltpu.VMEM),
            pl.BlockSpec(memory_space=pltpu.VMEM),
        ],
        out_specs=pl.BlockSpec(memory_space=pltpu.VMEM),
        scratch_shapes=[
            pltpu.VMEM((K_SLOTS, D), jnp.bfloat16),
            pltpu.VMEM((K_SLOTS, D), jnp.bfloat16),
            pltpu.SemaphoreType.DMA((N_CHUNKS,)),
            pltpu.SemaphoreType.DMA((N_CHUNKS,)),
        ],
        compiler_params=pltpu.CompilerParams(collective_id=0),
    )(ids2, ids_row, E)
